# baseline (speedup 1.0000x reference)
import functools

import jax
import jax.numpy as jnp
from jax import lax
from jax.experimental import pallas as pl
from jax.experimental.pallas import tpu as pltpu
from jax.experimental.pallas import tpu_sc as plsc

B, SEQ, D = 32, 196, 768
POOL, LENGTH, TOPK = 100, 10, 5
CW = 128
NCH = D // CW
NLC = CW // 16


def _mean_kernel(x_key_ref, prompt_ref, prompt_key_ref, mean_ref, rs_ref):
    xk = x_key_ref[...]
    xk = xk / jnp.maximum(
        jnp.sqrt(jnp.sum(xk * xk, axis=1, keepdims=True)), 1e-12)
    pk = prompt_key_ref[...]
    pk = pk / jnp.maximum(
        jnp.sqrt(jnp.sum(pk * pk, axis=1, keepdims=True)), 1e-12)

    sim = jnp.dot(xk, pk.T, preferred_element_type=jnp.float32)
    m = jnp.max(sim, axis=1, keepdims=True)
    e = jnp.exp(sim - m)
    p = e / jnp.sum(e, axis=1, keepdims=True)

    for l in range(LENGTH):
        mean_ref[:, l, :] = jnp.dot(
            p, prompt_ref[:, l, :],
            preferred_element_type=jnp.float32) * (1.0 / POOL)

    iota = jax.lax.broadcasted_iota(jnp.int32, (B, POOL), 1)
    v = sim
    total = jnp.float32(0.0)
    for _ in range(TOPK):
        mx = jnp.max(v, axis=1, keepdims=True)
        idx = jnp.min(jnp.where(v >= mx, iota, jnp.int32(POOL)),
                      axis=1, keepdims=True)
        total = total + jnp.sum(mx)
        v = jnp.where(iota == idx, -jnp.inf, v)
    rs_ref[...] = jnp.full((1, 1), total * (1.0 / B), jnp.float32)


_sc_mesh = plsc.VectorSubcoreMesh(core_axis_name="c", subcore_axis_name="s")


@functools.partial(
    pl.kernel,
    out_type=jax.ShapeDtypeStruct((B, LENGTH + SEQ, D), jnp.float32),
    mesh=_sc_mesh,
    scratch_types=[
        pltpu.VMEM((SEQ, CW), jnp.float32),
        pltpu.VMEM((LENGTH + SEQ, CW), jnp.float32),
        pltpu.VMEM((LENGTH, CW), jnp.float32),
    ],
    compiler_params=pltpu.CompilerParams(use_tc_tiling_on_sc=True),
)
def _sc_assemble(x_hbm, mean_hbm, out_hbm, xbuf, obuf, mbuf):
    wid = lax.axis_index("s") * 2 + lax.axis_index("c")
    for c in range(NCH):
        cols = pl.ds(c * CW, CW)
        pltpu.sync_copy(x_hbm.at[wid, :, cols], xbuf)
        pltpu.sync_copy(mean_hbm.at[wid, :, cols], mbuf)
        for l in range(LENGTH):
            for k in range(NLC):
                obuf[l, pl.ds(k * 16, 16)] = mbuf[l, pl.ds(k * 16, 16)]

        def _row(r, carry):
            for k in range(NLC):
                obuf[r + LENGTH, pl.ds(k * 16, 16)] = xbuf[r, pl.ds(k * 16, 16)]
            return carry

        lax.fori_loop(0, SEQ, _row, 0)
        pltpu.sync_copy(obuf, out_hbm.at[wid, :, cols])


@jax.jit
def kernel(x_embed, x_key, prompt, prompt_key):
    mean, rs = pl.pallas_call(
        _mean_kernel,
        in_specs=[
            pl.BlockSpec(memory_space=pltpu.MemorySpace.VMEM),
            pl.BlockSpec(memory_space=pltpu.MemorySpace.VMEM),
            pl.BlockSpec(memory_space=pltpu.MemorySpace.VMEM),
        ],
        out_specs=[
            pl.BlockSpec(memory_space=pltpu.MemorySpace.VMEM),
            pl.BlockSpec(memory_space=pltpu.MemorySpace.VMEM),
        ],
        out_shape=[
            jax.ShapeDtypeStruct((B, LENGTH, D), jnp.float32),
            jax.ShapeDtypeStruct((1, 1), jnp.float32),
        ],
    )(x_key, prompt, prompt_key)
    out = _sc_assemble(x_embed, mean)
    return out, rs[0, 0]


# speedup vs baseline: 1.5029x; 1.5029x over previous
import jax
import jax.numpy as jnp
from jax.experimental import pallas as pl
from jax.experimental.pallas import tpu as pltpu

B, SEQ, D = 32, 196, 768
POOL, LENGTH, TOPK = 100, 10, 5
CS = 2
NCK = B // CS
DEPTH = 4


def _fused_kernel(x_hbm, x_key_ref, prompt_ref, prompt_key_ref,
                  out_hbm, rs_ref, mean_s,
                  xb0, xb1, xb2, xb3, ob0, ob1, ob2, ob3, lsem, ssem):
    xbufs = (xb0, xb1, xb2, xb3)
    obufs = (ob0, ob1, ob2, ob3)

    def load(c, slot):
        return pltpu.make_async_copy(
            x_hbm.at[pl.ds(c * CS, CS), :, :], xbufs[slot], lsem.at[slot])

    def store(c, slot):
        return pltpu.make_async_copy(
            obufs[slot], out_hbm.at[pl.ds(c * CS, CS), :, :], ssem.at[slot])

    for c in range(DEPTH):
        load(c, c % DEPTH).start()

    xk = x_key_ref[...]
    xk = xk / jnp.maximum(
        jnp.sqrt(jnp.sum(xk * xk, axis=1, keepdims=True)), 1e-12)
    pk = prompt_key_ref[...]
    pk = pk / jnp.maximum(
        jnp.sqrt(jnp.sum(pk * pk, axis=1, keepdims=True)), 1e-12)

    sim = jnp.dot(xk, pk.T, preferred_element_type=jnp.float32)
    m = jnp.max(sim, axis=1, keepdims=True)
    e = jnp.exp(sim - m)
    p = e / jnp.sum(e, axis=1, keepdims=True)

    for l in range(LENGTH):
        mean_s[:, l, :] = jnp.dot(
            p, prompt_ref[:, l, :],
            preferred_element_type=jnp.float32) * (1.0 / POOL)

    iota = jax.lax.broadcasted_iota(jnp.int32, (B, POOL), 1)
    v = sim
    total = jnp.float32(0.0)
    for _ in range(TOPK):
        mx = jnp.max(v, axis=1, keepdims=True)
        idx = jnp.min(jnp.where(v >= mx, iota, jnp.int32(POOL)),
                      axis=1, keepdims=True)
        total = total + jnp.sum(mx)
        v = jnp.where(iota == idx, -jnp.inf, v)
    rs_ref[...] = jnp.full((1, 1), total * (1.0 / B), jnp.float32)

    for c in range(NCK):
        slot = c % DEPTH
        load(c, slot).wait()
        if c >= DEPTH:
            store(c - DEPTH, slot).wait()
        ob = obufs[slot]
        ob[:, LENGTH:, :] = xbufs[slot][...]
        ob[:, :LENGTH, :] = mean_s[c * CS:(c + 1) * CS, :, :]
        store(c, slot).start()
        if c + DEPTH < NCK:
            load(c + DEPTH, slot).start()

    for c in range(NCK - DEPTH, NCK):
        store(c, c % DEPTH).wait()


@jax.jit
def kernel(x_embed, x_key, prompt, prompt_key):
    obuf_t = pltpu.VMEM((CS, LENGTH + SEQ, D), jnp.float32)
    xbuf_t = pltpu.VMEM((CS, SEQ, D), jnp.float32)
    out, rs = pl.pallas_call(
        _fused_kernel,
        in_specs=[
            pl.BlockSpec(memory_space=pl.ANY),
            pl.BlockSpec(memory_space=pltpu.MemorySpace.VMEM),
            pl.BlockSpec(memory_space=pltpu.MemorySpace.VMEM),
            pl.BlockSpec(memory_space=pltpu.MemorySpace.VMEM),
        ],
        out_specs=[
            pl.BlockSpec(memory_space=pl.ANY),
            pl.BlockSpec(memory_space=pltpu.MemorySpace.VMEM),
        ],
        out_shape=[
            jax.ShapeDtypeStruct((B, LENGTH + SEQ, D), jnp.float32),
            jax.ShapeDtypeStruct((1, 1), jnp.float32),
        ],
        scratch_shapes=[
            pltpu.VMEM((B, LENGTH, D), jnp.float32),
            xbuf_t, xbuf_t, xbuf_t, xbuf_t,
            obuf_t, obuf_t, obuf_t, obuf_t,
            pltpu.SemaphoreType.DMA((DEPTH,)),
            pltpu.SemaphoreType.DMA((DEPTH,)),
        ],
    )(x_embed, x_key, prompt, prompt_key)
    return out, rs[0, 0]
